# baseline (speedup 1.0000x reference)
import functools

import jax
import jax.numpy as jnp
from jax import lax
from jax.experimental import pallas as pl
from jax.experimental.pallas import tpu as pltpu
from jax.experimental.pallas import tpu_sc as plsc

SEQ = 200
BATCH = 4096
EMB = 64
OUT = 16
LANES = 16
NCORES = 2
NSUB = 16
NW = NCORES * NSUB
BPW = BATCH // NW
NBUF = 4
UNROLL = 8


def _sc_pool(x, table):
    mesh = plsc.VectorSubcoreMesh(core_axis_name="c", subcore_axis_name="s")

    @functools.partial(
        pl.kernel,
        out_type=jax.ShapeDtypeStruct((BATCH, EMB), jnp.float32),
        mesh=mesh,
        scratch_types=[
            pltpu.VMEM((SEQ, BPW), jnp.int32),
            [pltpu.VMEM((BPW, EMB), jnp.float32) for _ in range(NBUF)],
            pltpu.VMEM((BPW, EMB), jnp.float32),
            [pltpu.SemaphoreType.DMA for _ in range(NBUF)],
        ],
        compiler_params=pltpu.CompilerParams(use_tc_tiling_on_sc=False),
    )
    def pool_kernel(x_hbm, tab_hbm, out_hbm, idx_v, rows, acc_v, sems):
        cid = lax.axis_index("c")
        sid = lax.axis_index("s")
        wid = sid * NCORES + cid
        base = wid * BPW

        pltpu.sync_copy(x_hbm.at[:, pl.ds(base, BPW)], idx_v)

        def gather(s, b):
            return pltpu.make_async_copy(tab_hbm.at[idx_v.at[s]], rows[b], sems[b])

        for b in range(NBUF):
            gather(b, b).start()

        zero = jnp.zeros((LANES,), jnp.float32)

        def zero_body(r, _):
            for c in range(EMB // LANES):
                acc_v[r, pl.ds(c * LANES, LANES)] = zero
            return 0

        lax.fori_loop(0, BPW, zero_body, 0)

        def accum(buf):
            def body(i, _):
                r0 = i * UNROLL
                for u in range(UNROLL):
                    for c in range(EMB // LANES):
                        sl = pl.ds(c * LANES, LANES)
                        plsc.addupdate(acc_v.at[r0 + u, sl], buf[r0 + u, sl])
                return 0

            lax.fori_loop(0, BPW // UNROLL, body, 0)

        def outer(g, _):
            for b in range(NBUF):
                s = g * NBUF + b
                gather(s, b).wait()
                accum(rows[b])
                ns = s + NBUF

                @pl.when(ns < SEQ)
                def _():
                    gather(ns, b).start()

            return 0

        lax.fori_loop(0, SEQ // NBUF, outer, 0)

        inv = jnp.float32(1.0 / SEQ)

        def scale_body(r, _):
            for c in range(EMB // LANES):
                sl = pl.ds(c * LANES, LANES)
                acc_v[r, sl] = acc_v[r, sl] * inv
            return 0

        lax.fori_loop(0, BPW, scale_body, 0)
        pltpu.sync_copy(acc_v, out_hbm.at[pl.ds(base, BPW), :])

    return pool_kernel(x, table)


def _tc_head(pooled, W, b2d):
    blk = 512

    def head_kernel(p_ref, w_ref, b_ref, o_ref):
        logits = jnp.dot(p_ref[...], w_ref[...],
                         preferred_element_type=jnp.float32) + b_ref[...]
        m = jnp.max(logits, axis=1, keepdims=True)
        z = logits - m
        lse = jnp.log(jnp.sum(jnp.exp(z), axis=1, keepdims=True))
        o_ref[...] = z - lse

    return pl.pallas_call(
        head_kernel,
        out_shape=jax.ShapeDtypeStruct((BATCH, OUT), jnp.float32),
        grid=(BATCH // blk,),
        in_specs=[
            pl.BlockSpec((blk, EMB), lambda i: (i, 0)),
            pl.BlockSpec((EMB, OUT), lambda i: (0, 0)),
            pl.BlockSpec((1, OUT), lambda i: (0, 0)),
        ],
        out_specs=pl.BlockSpec((blk, OUT), lambda i: (i, 0)),
    )(pooled, W, b2d)


def kernel(x, table, W, b):
    pooled = _sc_pool(x, table)
    return _tc_head(pooled, W, b.reshape(1, OUT))
